# baseline (speedup 1.0000x reference)
import jax
import jax.numpy as jnp
from jax.experimental import pallas as pl
from jax.experimental.pallas import tpu as pltpu
from jax.experimental.pallas import tpu_sc as plsc

IN_CH = 64
HIDDEN = 64
GATHER_WINDOW = 128


def _sc_gather(table, idx_2d):
    n = idx_2d.shape[1]
    mesh = plsc.VectorSubcoreMesh(core_axis_name="core", subcore_axis_name="subcore")

    @pl.kernel(
        out_type=jax.ShapeDtypeStruct((n, IN_CH), table.dtype),
        mesh=mesh,
        compiler_params=pltpu.CompilerParams(use_tc_tiling_on_sc=False),
    )
    def gather_kernel(table_hbm, i_hbm, o_hbm):
        def body(i_vmem, o_vmem):
            pltpu.sync_copy(table_hbm.at[i_vmem.at[0]], o_vmem)

        pltpu.emit_pipeline(
            body,
            grid=(n // GATHER_WINDOW,),
            in_specs=[pl.BlockSpec((1, GATHER_WINDOW), index_map=lambda i: (0, i))],
            out_specs=[pl.BlockSpec((GATHER_WINDOW, IN_CH), index_map=lambda i: (i, 0))],
            core_axis_name=("core", "subcore"),
            dimension_semantics=(pltpu.PARALLEL,),
        )(i_hbm, o_hbm)

    return gather_kernel(table, idx_2d)


def _tc_project(emb3, w16, bias_col):
    L, B, _ = emb3.shape

    def body(e_ref, w_ref, b_ref, o_ref):
        prod = jax.lax.dot_general(
            w_ref[...],
            e_ref[0],
            dimension_numbers=(((1,), (1,)), ((), ())),
            preferred_element_type=jnp.float32,
        )
        o_ref[0] = prod + b_ref[...]

    return pl.pallas_call(
        body,
        grid=(L,),
        in_specs=[
            pl.BlockSpec((1, B, IN_CH), lambda i: (i, 0, 0)),
            pl.BlockSpec((HIDDEN, IN_CH), lambda i: (0, 0)),
            pl.BlockSpec((HIDDEN, 1), lambda i: (0, 0)),
        ],
        out_specs=pl.BlockSpec((1, HIDDEN, B), lambda i: (i, 0, 0)),
        out_shape=jax.ShapeDtypeStruct((L, HIDDEN, B), jnp.float32),
    )(emb3, w16, bias_col)


def kernel(x, wtb, W, b):
    B, L = x.shape
    n = B * L
    table16 = wtb.astype(jnp.bfloat16)
    idx_2d = x.T.reshape(1, n)
    emb = _sc_gather(table16, idx_2d)
    emb3 = emb.reshape(L, B, IN_CH)
    out_phys = _tc_project(emb3, W.astype(jnp.bfloat16), b.reshape(HIDDEN, 1))
    return jnp.transpose(out_phys, (2, 0, 1))


# speedup vs baseline: 1.4101x; 1.4101x over previous
import jax
import jax.numpy as jnp
from jax.experimental import pallas as pl
from jax.experimental.pallas import tpu as pltpu
from jax.experimental.pallas import tpu_sc as plsc

IN_CH = 64
HIDDEN = 64
GATHER_WINDOW = 128


def _sc_gather(table, idx_2d):
    n = idx_2d.shape[1]
    mesh = plsc.VectorSubcoreMesh(core_axis_name="core", subcore_axis_name="subcore")

    @pl.kernel(
        out_type=jax.ShapeDtypeStruct((n, IN_CH), table.dtype),
        mesh=mesh,
        compiler_params=pltpu.CompilerParams(use_tc_tiling_on_sc=False),
    )
    def gather_kernel(table_hbm, i_hbm, o_hbm):
        def body(i_vmem, o_vmem):
            pltpu.sync_copy(table_hbm.at[i_vmem.at[0]], o_vmem)

        pltpu.emit_pipeline(
            body,
            grid=(n // GATHER_WINDOW,),
            in_specs=[pl.BlockSpec((1, GATHER_WINDOW), index_map=lambda i: (0, i))],
            out_specs=[pl.BlockSpec((GATHER_WINDOW, IN_CH), index_map=lambda i: (i, 0))],
            core_axis_name=("core", "subcore"),
            dimension_semantics=(pltpu.PARALLEL,),
        )(i_hbm, o_hbm)

    return gather_kernel(table, idx_2d)


def _tc_project(emb3, w, bias_col):
    L, HB, _ = emb3.shape
    B = 2 * HB

    def body(e_ref, w_ref, b_ref, o_ref):
        e = e_ref[0]
        dn = (((1,), (1,)), ((), ()))
        lo = jax.lax.dot_general(
            w_ref[...], e[:, 0:IN_CH], dn, preferred_element_type=jnp.float32
        )
        hi = jax.lax.dot_general(
            w_ref[...], e[:, IN_CH : 2 * IN_CH], dn,
            preferred_element_type=jnp.float32,
        )
        o_ref[0, :, 0:HB] = lo + b_ref[...]
        o_ref[0, :, HB:B] = hi + b_ref[...]

    return pl.pallas_call(
        body,
        grid=(L,),
        in_specs=[
            pl.BlockSpec((1, HB, 2 * IN_CH), lambda i: (i, 0, 0)),
            pl.BlockSpec((HIDDEN, IN_CH), lambda i: (0, 0)),
            pl.BlockSpec((HIDDEN, 1), lambda i: (0, 0)),
        ],
        out_specs=pl.BlockSpec((1, HIDDEN, B), lambda i: (i, 0, 0)),
        out_shape=jax.ShapeDtypeStruct((L, HIDDEN, B), jnp.float32),
    )(emb3, w, bias_col)


def kernel(x, wtb, W, b):
    B, L = x.shape
    n = B * L
    HB = B // 2
    idx = x.T.reshape(L, 2, HB).transpose(0, 2, 1).reshape(1, n)
    emb = _sc_gather(wtb, idx)
    emb3 = emb.reshape(L, HB, 2 * IN_CH)
    out_phys = _tc_project(emb3, W, b.reshape(HIDDEN, 1))
    return jnp.transpose(out_phys, (2, 0, 1))


# speedup vs baseline: 1.7374x; 1.2321x over previous
import jax
import jax.numpy as jnp
from jax.experimental import pallas as pl
from jax.experimental.pallas import tpu as pltpu
from jax.experimental.pallas import tpu_sc as plsc

IN_CH = 64
HIDDEN = 64
GATHER_WINDOW = 128
RP_BLK = 2048
RP_GRID = 245
SPLIT = RP_BLK * RP_GRID


def _tc_repack(wtb_t):
    n_col_blocks = wtb_t.shape[1] // RP_BLK

    def body(lo_ref, hi_ref, o_ref):
        o_ref[:, 0:IN_CH] = jnp.transpose(lo_ref[...], (1, 0))
        o_ref[:, IN_CH : 2 * IN_CH] = jnp.transpose(hi_ref[...], (1, 0))

    return pl.pallas_call(
        body,
        grid=(RP_GRID,),
        in_specs=[
            pl.BlockSpec((IN_CH, RP_BLK), lambda i: (0, i)),
            pl.BlockSpec(
                (IN_CH, RP_BLK), lambda i: (0, jnp.minimum(i + RP_GRID, n_col_blocks))
            ),
        ],
        out_specs=pl.BlockSpec((RP_BLK, 2 * IN_CH), lambda i: (i, 0)),
        out_shape=jax.ShapeDtypeStruct((SPLIT, 2 * IN_CH), jnp.float32),
    )(wtb_t, wtb_t)


def _sc_gather(table2, idx_2d):
    n = idx_2d.shape[1]
    mesh = plsc.VectorSubcoreMesh(core_axis_name="core", subcore_axis_name="subcore")

    @pl.kernel(
        out_type=jax.ShapeDtypeStruct((n, 2 * IN_CH), jnp.float32),
        mesh=mesh,
        compiler_params=pltpu.CompilerParams(use_tc_tiling_on_sc=False),
    )
    def gather_kernel(table_hbm, i_hbm, o_hbm):
        def body(i_vmem, o_vmem):
            pltpu.sync_copy(table_hbm.at[i_vmem.at[0]], o_vmem)

        pltpu.emit_pipeline(
            body,
            grid=(n // GATHER_WINDOW,),
            in_specs=[pl.BlockSpec((1, GATHER_WINDOW), index_map=lambda i: (0, i))],
            out_specs=[
                pl.BlockSpec((GATHER_WINDOW, 2 * IN_CH), index_map=lambda i: (i, 0))
            ],
            core_axis_name=("core", "subcore"),
            dimension_semantics=(pltpu.PARALLEL,),
        )(i_hbm, o_hbm)

    return gather_kernel(table2, idx_2d)


def _tc_project(emb3, x_t, w, bias_col):
    L, B, _ = emb3.shape

    def body(e_ref, i_ref, w_ref, b_ref, o_ref):
        e = e_ref[0]
        dn = (((1,), (1,)), ((), ()))
        lo = jax.lax.dot_general(
            w_ref[...], e[:, 0:IN_CH], dn, preferred_element_type=jnp.float32
        )
        hi = jax.lax.dot_general(
            w_ref[...], e[:, IN_CH : 2 * IN_CH], dn,
            preferred_element_type=jnp.float32,
        )
        hi_sel = i_ref[0] >= SPLIT
        o_ref[0] = jnp.where(hi_sel, hi, lo) + b_ref[...]

    return pl.pallas_call(
        body,
        grid=(L,),
        in_specs=[
            pl.BlockSpec((1, B, 2 * IN_CH), lambda i: (i, 0, 0)),
            pl.BlockSpec((1, 1, B), lambda i: (i, 0, 0)),
            pl.BlockSpec((HIDDEN, IN_CH), lambda i: (0, 0)),
            pl.BlockSpec((HIDDEN, 1), lambda i: (0, 0)),
        ],
        out_specs=pl.BlockSpec((1, HIDDEN, B), lambda i: (i, 0, 0)),
        out_shape=jax.ShapeDtypeStruct((L, HIDDEN, B), jnp.float32),
    )(emb3, x_t, w, bias_col)


def kernel(x, wtb, W, b):
    B, L = x.shape
    n = B * L
    table2 = _tc_repack(wtb.T)
    x_t = x.T
    idx_2d = jnp.where(x_t >= SPLIT, x_t - SPLIT, x_t).reshape(1, n)
    emb = _sc_gather(table2, idx_2d)
    emb3 = emb.reshape(L, B, 2 * IN_CH)
    out_phys = _tc_project(emb3, x_t.reshape(L, 1, B), W, b.reshape(HIDDEN, 1))
    return jnp.transpose(out_phys, (2, 0, 1))
